# baseline (speedup 1.0000x reference)
import functools

import jax
import jax.numpy as jnp
from jax import lax
from jax.experimental import pallas as pl
from jax.experimental.pallas import tpu as pltpu
from jax.experimental.pallas import tpu_sc as plsc

Q = 1024
N = 100000
NPAD = 102400
KB = 4096
NKB = NPAD // KB
G = 128
NCH = NPAD // G
CPB = KB // G
TOPK = 10
QB = 256
PADB2 = 3.3e29


def _dist_body(x2_ref, a2_ref, keys_ref, b2_ref, s_ref, cm_ref):
    ab2 = lax.dot_general(
        x2_ref[...], keys_ref[...], (((1,), (1,)), ((), ())),
        preferred_element_type=jnp.float32)
    s = (a2_ref[...] + b2_ref[0]) - ab2
    col = (pl.program_id(0) * KB
           + lax.broadcasted_iota(jnp.int32, s.shape, 1))
    s = jnp.where(col < N, s, jnp.float32(PADB2))
    mins = []
    for c in range(CPB):
        sl = s[:, c * G:(c + 1) * G]
        s_ref[c] = sl
        mins.append(jnp.min(sl, axis=1, keepdims=True))
    cm_ref[0] = jnp.concatenate(mins, axis=1)


def _select_body(cm_ref, csel_ref, fidx_ref):
    cm = jnp.concatenate(
        [cm_ref[i] for i in range(NKB)], axis=1)
    col = lax.broadcasted_iota(jnp.int32, cm.shape, 1)
    picks = []
    for _ in range(TOPK):
        m = jnp.min(cm, axis=1, keepdims=True)
        c_r = jnp.min(jnp.where(cm == m, col, NCH), axis=1, keepdims=True)
        cm = jnp.where(col == c_r, jnp.float32(jnp.inf), cm)
        picks.append(c_r)
    csel = jnp.concatenate(picks, axis=1)
    csel_ref[...] = csel
    qrow = lax.broadcasted_iota(jnp.int32, (Q, TOPK), 0)
    fidx_ref[...] = csel * Q + qrow


def _final_body(gs_ref, gv_ref, csel_ref, out_ref):
    s = gs_ref[...]
    v = gv_ref[...]
    csel = csel_ref[...]
    off = lax.broadcasted_iota(jnp.int32, (QB, G), 1)
    gidx = jnp.concatenate(
        [csel[:, r:r + 1] * G + off for r in range(TOPK)], axis=1)
    big_i = jnp.int32(2 ** 30)
    vals10 = []
    for _ in range(TOPK):
        m = jnp.min(s, axis=1, keepdims=True)
        i_star = jnp.min(jnp.where(s == m, gidx, big_i), axis=1, keepdims=True)
        hit = gidx == i_star
        vals10.append(jnp.min(jnp.where(hit, v, big_i), axis=1))
        s = jnp.where(hit, jnp.float32(jnp.inf), s)
    rank = None
    for i in range(TOPK):
        cnt = None
        for j in range(TOPK):
            e = (vals10[i] == vals10[j]).astype(jnp.int32)
            cnt = e if cnt is None else cnt + e
        r_i = cnt * 2048 - vals10[i]
        rank = r_i if rank is None else jnp.maximum(rank, r_i)
    out_ref[0, 0] = (2048 - (rank & 2047)) & 2047


def _make_gather():
    nc, ns = 2, 16
    nw = nc * ns
    b = Q * TOPK
    bpw = b // nw
    ch = 64
    nch_loop = bpw // ch
    mesh = plsc.VectorSubcoreMesh(core_axis_name="c", subcore_axis_name="s")

    @functools.partial(
        pl.kernel, mesh=mesh,
        out_type=(
            jax.ShapeDtypeStruct((b, G), jnp.float32),
            jax.ShapeDtypeStruct((b, G), jnp.int32),
        ),
        scratch_types=[
            pltpu.VMEM((bpw,), jnp.int32),
            pltpu.VMEM((bpw,), jnp.int32),
            pltpu.VMEM((ch, G), jnp.float32),
            pltpu.VMEM((ch, G), jnp.float32),
            pltpu.VMEM((ch, G), jnp.int32),
            pltpu.VMEM((ch, G), jnp.int32),
            pltpu.SemaphoreType.DMA,
            pltpu.SemaphoreType.DMA,
            pltpu.SemaphoreType.DMA,
            pltpu.SemaphoreType.DMA,
        ],
    )
    def gather(s_hbm, fidx_hbm, cidx_hbm, vtab_hbm, gs_hbm, gv_hbm,
               fidx_v, cidx_v, s0, s1, v0, v1, g0, g1, w0, w1):
        wid = lax.axis_index("s") * nc + lax.axis_index("c")
        base = wid * bpw
        pltpu.sync_copy(fidx_hbm.at[pl.ds(base, bpw)], fidx_v)
        pltpu.sync_copy(cidx_hbm.at[pl.ds(base, bpw)], cidx_v)
        sbuf, vbuf, gsem, wsem = (s0, s1), (v0, v1), (g0, g1), (w0, w1)
        gpend, wpend = {}, {}
        for j in range(nch_loop + 1):
            if j < nch_loop:
                p = j % 2
                if j >= 2:
                    for c in wpend.pop(j - 2):
                        c.wait()
                gpend[j] = (
                    pltpu.async_copy(
                        s_hbm.at[fidx_v.at[pl.ds(j * ch, ch)]],
                        sbuf[p], gsem[p]),
                    pltpu.async_copy(
                        vtab_hbm.at[cidx_v.at[pl.ds(j * ch, ch)]],
                        vbuf[p], gsem[p]),
                )
            if j >= 1:
                jj = j - 1
                p = jj % 2
                for c in gpend.pop(jj):
                    c.wait()
                wpend[jj] = (
                    pltpu.async_copy(
                        sbuf[p], gs_hbm.at[pl.ds(base + jj * ch, ch)], wsem[p]),
                    pltpu.async_copy(
                        vbuf[p], gv_hbm.at[pl.ds(base + jj * ch, ch)], wsem[p]),
                )
        for jj in (nch_loop - 2, nch_loop - 1):
            for c in wpend.pop(jj):
                c.wait()

    return gather


def kernel(x, ver, keys, vals):
    del ver
    a2 = jnp.sum(x * x, axis=1, keepdims=True)
    b2 = jnp.sum(keys * keys, axis=1)
    x2 = x + x
    b2_p = jnp.concatenate(
        [b2, jnp.full((NPAD - N,), PADB2, jnp.float32)]).reshape(NKB, 1, KB)
    vals_p = jnp.pad(vals, (0, NPAD - N)).reshape(NCH, G)

    s_mat, cm_blk = pl.pallas_call(
        _dist_body,
        grid=(NKB,),
        in_specs=[
            pl.BlockSpec((Q, 128), lambda i: (0, 0)),
            pl.BlockSpec((Q, 1), lambda i: (0, 0)),
            pl.BlockSpec((KB, 128), lambda i: (i, 0)),
            pl.BlockSpec((1, 1, KB), lambda i: (i, 0, 0)),
        ],
        out_specs=[
            pl.BlockSpec((CPB, Q, G), lambda i: (i, 0, 0)),
            pl.BlockSpec((1, Q, CPB), lambda i: (i, 0, 0)),
        ],
        out_shape=[
            jax.ShapeDtypeStruct((NCH, Q, G), jnp.float32),
            jax.ShapeDtypeStruct((NKB, Q, CPB), jnp.float32),
        ],
    )(x2, a2, keys, b2_p)

    csel, fidx = pl.pallas_call(
        _select_body,
        out_shape=[
            jax.ShapeDtypeStruct((Q, TOPK), jnp.int32),
            jax.ShapeDtypeStruct((Q, TOPK), jnp.int32),
        ],
    )(cm_blk)

    fidx_j = fidx.reshape(Q // 8, 8, TOPK).transpose(0, 2, 1).reshape(-1)
    cidx_j = csel.reshape(Q // 8, 8, TOPK).transpose(0, 2, 1).reshape(-1)
    gs, gv = _make_gather()(
        s_mat.reshape(NCH * Q, G), fidx_j, cidx_j, vals_p)

    out = pl.pallas_call(
        _final_body,
        grid=(Q // QB,),
        in_specs=[
            pl.BlockSpec((QB, TOPK * G), lambda i: (i, 0)),
            pl.BlockSpec((QB, TOPK * G), lambda i: (i, 0)),
            pl.BlockSpec((QB, TOPK), lambda i: (i, 0)),
        ],
        out_specs=pl.BlockSpec((1, 1, QB), lambda i: (i, 0, 0)),
        out_shape=jax.ShapeDtypeStruct((Q // QB, 1, QB), jnp.int32),
    )(gs.reshape(Q, TOPK * G), gv.reshape(Q, TOPK * G), csel)
    return out.reshape(Q)


# speedup vs baseline: 1.0116x; 1.0116x over previous
import functools

import jax
import jax.numpy as jnp
from jax import lax
from jax.experimental import pallas as pl
from jax.experimental.pallas import tpu as pltpu
from jax.experimental.pallas import tpu_sc as plsc

Q = 1024
N = 100000
NPAD = 102400
KB = 5120
NKB = NPAD // KB
G = 128
NCH = NPAD // G
CPB = KB // G
TOPK = 10
QB = 256
PADB2 = 3.3e29


def _dist_body(x2_ref, a2_ref, keys_ref, b2_ref, s_ref, cm_ref):
    ab2 = lax.dot_general(
        x2_ref[...], keys_ref[...], (((1,), (1,)), ((), ())),
        preferred_element_type=jnp.float32)
    s = (a2_ref[...] + b2_ref[0]) - ab2
    col = (pl.program_id(0) * KB
           + lax.broadcasted_iota(jnp.int32, s.shape, 1))
    s = jnp.where(col < N, s, jnp.float32(PADB2))
    mins = []
    for c in range(CPB):
        sl = s[:, c * G:(c + 1) * G]
        s_ref[c] = sl
        mins.append(jnp.min(sl, axis=1, keepdims=True))
    cm_ref[0] = jnp.concatenate(mins, axis=1)


def _select_body(cm_ref, csel_ref, fidx_ref):
    cm = jnp.concatenate(
        [cm_ref[i] for i in range(NKB)], axis=1)
    col = lax.broadcasted_iota(jnp.int32, cm.shape, 1)
    picks = []
    for _ in range(TOPK):
        m = jnp.min(cm, axis=1, keepdims=True)
        c_r = jnp.min(jnp.where(cm == m, col, NCH), axis=1, keepdims=True)
        cm = jnp.where(col == c_r, jnp.float32(jnp.inf), cm)
        picks.append(c_r)
    csel = jnp.concatenate(picks, axis=1)
    csel_ref[...] = csel
    qrow = lax.broadcasted_iota(jnp.int32, (Q, TOPK), 0)
    fidx_ref[...] = csel * Q + qrow


def _final_body(gs_ref, gv_ref, csel_ref, out_ref):
    s = gs_ref[...]
    v = gv_ref[...]
    csel = csel_ref[...]
    off = lax.broadcasted_iota(jnp.int32, (QB, G), 1)
    gidx = jnp.concatenate(
        [csel[:, r:r + 1] * G + off for r in range(TOPK)], axis=1)
    big_i = jnp.int32(2 ** 30)
    vals10 = []
    for _ in range(TOPK):
        m = jnp.min(s, axis=1, keepdims=True)
        i_star = jnp.min(jnp.where(s == m, gidx, big_i), axis=1, keepdims=True)
        hit = gidx == i_star
        vals10.append(jnp.min(jnp.where(hit, v, big_i), axis=1))
        s = jnp.where(hit, jnp.float32(jnp.inf), s)
    rank = None
    for i in range(TOPK):
        cnt = None
        for j in range(TOPK):
            e = (vals10[i] == vals10[j]).astype(jnp.int32)
            cnt = e if cnt is None else cnt + e
        r_i = cnt * 2048 - vals10[i]
        rank = r_i if rank is None else jnp.maximum(rank, r_i)
    out_ref[0, 0] = (2048 - (rank & 2047)) & 2047


def _make_gather():
    nc, ns = 2, 16
    nw = nc * ns
    b = Q * TOPK
    bpw = b // nw
    ch = 64
    nch_loop = bpw // ch
    mesh = plsc.VectorSubcoreMesh(core_axis_name="c", subcore_axis_name="s")

    @functools.partial(
        pl.kernel, mesh=mesh,
        out_type=(
            jax.ShapeDtypeStruct((b, G), jnp.float32),
            jax.ShapeDtypeStruct((b, G), jnp.int32),
        ),
        scratch_types=[
            pltpu.VMEM((bpw,), jnp.int32),
            pltpu.VMEM((bpw,), jnp.int32),
            pltpu.VMEM((ch, G), jnp.float32),
            pltpu.VMEM((ch, G), jnp.float32),
            pltpu.VMEM((ch, G), jnp.int32),
            pltpu.VMEM((ch, G), jnp.int32),
            pltpu.SemaphoreType.DMA,
            pltpu.SemaphoreType.DMA,
            pltpu.SemaphoreType.DMA,
            pltpu.SemaphoreType.DMA,
        ],
    )
    def gather(s_hbm, fidx_hbm, cidx_hbm, vtab_hbm, gs_hbm, gv_hbm,
               fidx_v, cidx_v, s0, s1, v0, v1, g0, g1, w0, w1):
        wid = lax.axis_index("s") * nc + lax.axis_index("c")
        base = wid * bpw
        pltpu.sync_copy(fidx_hbm.at[pl.ds(base, bpw)], fidx_v)
        pltpu.sync_copy(cidx_hbm.at[pl.ds(base, bpw)], cidx_v)
        sbuf, vbuf, gsem, wsem = (s0, s1), (v0, v1), (g0, g1), (w0, w1)
        gpend, wpend = {}, {}
        for j in range(nch_loop + 1):
            if j < nch_loop:
                p = j % 2
                if j >= 2:
                    for c in wpend.pop(j - 2):
                        c.wait()
                gpend[j] = (
                    pltpu.async_copy(
                        s_hbm.at[fidx_v.at[pl.ds(j * ch, ch)]],
                        sbuf[p], gsem[p]),
                    pltpu.async_copy(
                        vtab_hbm.at[cidx_v.at[pl.ds(j * ch, ch)]],
                        vbuf[p], gsem[p]),
                )
            if j >= 1:
                jj = j - 1
                p = jj % 2
                for c in gpend.pop(jj):
                    c.wait()
                wpend[jj] = (
                    pltpu.async_copy(
                        sbuf[p], gs_hbm.at[pl.ds(base + jj * ch, ch)], wsem[p]),
                    pltpu.async_copy(
                        vbuf[p], gv_hbm.at[pl.ds(base + jj * ch, ch)], wsem[p]),
                )
        for jj in (nch_loop - 2, nch_loop - 1):
            for c in wpend.pop(jj):
                c.wait()

    return gather


def kernel(x, ver, keys, vals):
    del ver
    a2 = jnp.sum(x * x, axis=1, keepdims=True)
    b2 = jnp.sum(keys * keys, axis=1)
    x2 = x + x
    b2_p = jnp.concatenate(
        [b2, jnp.full((NPAD - N,), PADB2, jnp.float32)]).reshape(NKB, 1, KB)
    vals_p = jnp.pad(vals, (0, NPAD - N)).reshape(NCH, G)

    s_mat, cm_blk = pl.pallas_call(
        _dist_body,
        grid=(NKB,),
        in_specs=[
            pl.BlockSpec((Q, 128), lambda i: (0, 0)),
            pl.BlockSpec((Q, 1), lambda i: (0, 0)),
            pl.BlockSpec((KB, 128), lambda i: (i, 0)),
            pl.BlockSpec((1, 1, KB), lambda i: (i, 0, 0)),
        ],
        out_specs=[
            pl.BlockSpec((CPB, Q, G), lambda i: (i, 0, 0)),
            pl.BlockSpec((1, Q, CPB), lambda i: (i, 0, 0)),
        ],
        out_shape=[
            jax.ShapeDtypeStruct((NCH, Q, G), jnp.float32),
            jax.ShapeDtypeStruct((NKB, Q, CPB), jnp.float32),
        ],
    )(x2, a2, keys, b2_p)

    csel, fidx = pl.pallas_call(
        _select_body,
        out_shape=[
            jax.ShapeDtypeStruct((Q, TOPK), jnp.int32),
            jax.ShapeDtypeStruct((Q, TOPK), jnp.int32),
        ],
    )(cm_blk)

    gs, gv = _make_gather()(
        s_mat.reshape(NCH * Q, G), fidx.reshape(Q * TOPK),
        csel.reshape(Q * TOPK), vals_p)

    out = pl.pallas_call(
        _final_body,
        grid=(Q // QB,),
        in_specs=[
            pl.BlockSpec((QB, TOPK * G), lambda i: (i, 0)),
            pl.BlockSpec((QB, TOPK * G), lambda i: (i, 0)),
            pl.BlockSpec((QB, TOPK), lambda i: (i, 0)),
        ],
        out_specs=pl.BlockSpec((1, 1, QB), lambda i: (i, 0, 0)),
        out_shape=jax.ShapeDtypeStruct((Q // QB, 1, QB), jnp.int32),
    )(gs.reshape(Q, TOPK * G), gv.reshape(Q, TOPK * G), csel)
    return out.reshape(Q)
